# baseline (speedup 1.0000x reference)
import functools

import jax
import jax.numpy as jnp
from jax import lax
from jax.experimental import pallas as pl
from jax.experimental.pallas import tpu as pltpu
from jax.experimental.pallas import tpu_sc as plsc

D_M = 64
SCALE = 8.0
NC = 2
NS = 16
NW = NC * NS
G = 128
K = 4
CH = K * G
IR = 2 * K


@functools.lru_cache(maxsize=None)
def _build(B):
    b_per_w = B // NW
    n_outer = b_per_w // (2 * CH)
    mesh = plsc.VectorSubcoreMesh(core_axis_name="c", subcore_axis_name="s")

    @functools.partial(
        pl.kernel,
        out_type=jax.ShapeDtypeStruct((B, D_M), jnp.float32),
        mesh=mesh,
        scratch_types=[
            pltpu.VMEM((IR, G), jnp.int32),
            pltpu.VMEM((2, CH, D_M), jnp.float32),
            pltpu.SemaphoreType.DMA,
        ],
        compiler_params=pltpu.CompilerParams(use_tc_tiling_on_sc=False),
    )
    def k(idx_hbm, table_hbm, out_hbm, idx_v, rows_v, gsem):
        wid = lax.axis_index("s") * NC + lax.axis_index("c")
        base = wid * b_per_w

        def outer_body(c, carry):
            r0 = pl.multiple_of(base // G + c * IR, IR)
            pltpu.sync_copy(idx_hbm.at[pl.ds(r0, IR)], idx_v)
            for half in range(2):
                cps = [
                    pltpu.async_copy(
                        table_hbm.at[idx_v.at[half * K + j]],
                        rows_v.at[half, pl.ds(j * G, G)],
                        gsem,
                    )
                    for j in range(K)
                ]
                for cp in cps:
                    cp.wait()

                def row_body(r, rcarry):
                    for s4 in range(D_M // 16):
                        sl = (half, r, pl.ds(s4 * 16, 16))
                        rows_v[sl] = rows_v[sl] + SCALE
                    return rcarry

                lax.fori_loop(0, CH, row_body, 0)
                pltpu.sync_copy(
                    rows_v.at[half],
                    out_hbm.at[pl.ds(base + c * 2 * CH + half * CH, CH)],
                )
            return carry

        lax.fori_loop(0, n_outer, outer_body, 0)

    return k


def kernel(x, table):
    s0, s1 = x.shape
    B = s0 * s1
    idx = x.reshape(B // G, G)
    out = _build(B)(idx, table)
    return out.reshape(s0, s1, D_M)
